# baseline (speedup 1.0000x reference)
import functools

import jax
import jax.numpy as jnp
from jax import lax
from jax.experimental import pallas as pl
from jax.experimental.pallas import tpu as pltpu
from jax.experimental.pallas import tpu_sc as plsc

B, V, C = 8, 50000, 16
H = W = 512
HW = H * W

NC, NS = 2, 16
NW = NC * NS
LANES = 16

PIX_PER_W = HW // NW
PIX_SUB = 4096
TBLK = 2000
TPITCH = TBLK + 1
N_TTASK = B * (V // TBLK)
TTASK_PER_W = (N_TTASK + NW - 1) // NW

ROWS = B * C
ROW_PER_W = ROWS // NW
PCH = 4096
N_PCH = HW // PCH

_mesh = plsc.VectorSubcoreMesh(core_axis_name="c", subcore_axis_name="s")


def _wid():
    return lax.axis_index("s") * NC + lax.axis_index("c")


def _prep_body(vv_hbm, ind_hbm, v2i_hbm, cidx_hbm, tblt_hbm,
               ind_v, v2i_v, cidx_v, tin_v, tcol_v):
    w = _wid()

    pltpu.sync_copy(ind_hbm, ind_v)
    for sub in range(PIX_PER_W // PIX_SUB):
        poff = w * PIX_PER_W + sub * PIX_SUB
        row0 = poff // W
        pltpu.sync_copy(v2i_hbm.at[pl.ds(row0, PIX_SUB // W), :], v2i_v)

        @plsc.parallel_loop(0, PIX_SUB // LANES, unroll=8)
        def _(j):
            idx = v2i_v[j // (W // LANES), pl.ds((j % (W // LANES)) * LANES,
                                                 LANES)]
            cidx_v[pl.ds(j * LANES, LANES)] = plsc.load_gather(ind_v, [idx])
        pltpu.sync_copy(cidx_v, cidx_hbm.at[pl.ds(poff, PIX_SUB)])

    iota = lax.iota(jnp.int32, LANES)

    def ttask(t):
        b = t // (V // TBLK)
        roff = (t % (V // TBLK)) * TBLK
        pltpu.sync_copy(vv_hbm.at[b, pl.ds(roff, TBLK), :], tin_v)

        @plsc.parallel_loop(0, TBLK, unroll=8)
        def _(p):
            val = tin_v[p, :]
            plsc.store_scatter(
                tcol_v, [iota, jnp.full((LANES,), p, jnp.int32)], val)
        pltpu.sync_copy(tcol_v.at[:, pl.ds(0, TBLK)],
                        tblt_hbm.at[pl.ds(b * C, C), pl.ds(roff, TBLK)])

    def touter(k, _):
        t = w + k * NW

        @pl.when(t < N_TTASK)
        def _():
            ttask(t)

        return 0

    lax.fori_loop(0, TTASK_PER_W, touter, 0)


def _gather_body(tblt_hbm, cidx_hbm, out_hbm,
                 ta_v, tb_v, c0_v, c1_v, o00_v, o10_v, o01_v, o11_v,
                 si0, si1, so0, so1):
    w = _wid()
    cbufs = ((c0_v, si0), (c1_v, si1))
    obufs = ((o00_v, o10_v, so0), (o01_v, o11_v, so1))

    for half in range(ROW_PER_W // 2):
        r0 = w * ROW_PER_W + half * 2
        pltpu.sync_copy(tblt_hbm.at[r0], ta_v)
        pltpu.sync_copy(tblt_hbm.at[r0 + 1], tb_v)
        pltpu.async_copy(cidx_hbm.at[pl.ds(0, PCH)], c0_v, si0)

        def sub(kk, i):
            cin, si = cbufs[i]
            oa, ob, so = obufs[i]
            cnx, snx = cbufs[1 - i]
            ch = kk * 2 + i
            pltpu.make_async_copy(cidx_hbm.at[pl.ds(0, PCH)], cin, si).wait()

            @pl.when(ch + 1 < N_PCH)
            def _():
                pltpu.async_copy(
                    cidx_hbm.at[pl.ds((ch + 1) * PCH, PCH)], cnx, snx)

            @pl.when(kk > 0)
            def _():
                pltpu.make_async_copy(
                    oa, out_hbm.at[r0, pl.ds(0, PCH)], so).wait()
                pltpu.make_async_copy(
                    ob, out_hbm.at[r0 + 1, pl.ds(0, PCH)], so).wait()

            @plsc.parallel_loop(0, PCH // LANES, unroll=8)
            def _(j):
                s = pl.ds(j * LANES, LANES)
                idx = cin[s]
                oa[s] = plsc.load_gather(ta_v, [idx])
                ob[s] = plsc.load_gather(tb_v, [idx])

            poff = ch * PCH
            pltpu.async_copy(oa, out_hbm.at[r0, pl.ds(poff, PCH)], so)
            pltpu.async_copy(ob, out_hbm.at[r0 + 1, pl.ds(poff, PCH)], so)

        def kk_body(kk, _):
            sub(kk, 0)
            sub(kk, 1)
            return 0

        lax.fori_loop(0, N_PCH // 2, kk_body, 0)
        for oa, ob, so in obufs:
            pltpu.make_async_copy(oa, out_hbm.at[r0, pl.ds(0, PCH)], so).wait()
            pltpu.make_async_copy(
                ob, out_hbm.at[r0 + 1, pl.ds(0, PCH)], so).wait()


_params = pltpu.CompilerParams(use_tc_tiling_on_sc=False,
                               needs_layout_passes=False)

_prep = functools.partial(
    pl.kernel,
    out_type=(
        jax.ShapeDtypeStruct((HW,), jnp.int32),
        jax.ShapeDtypeStruct((B * C, V), jnp.float32),
    ),
    mesh=_mesh,
    compiler_params=_params,
    scratch_types=[
        pltpu.VMEM((V,), jnp.int32),
        pltpu.VMEM((PIX_SUB // W, W), jnp.int32),
        pltpu.VMEM((PIX_SUB,), jnp.int32),
        pltpu.VMEM((TBLK, C), jnp.float32),
        pltpu.VMEM((C, TPITCH), jnp.float32),
    ],
)(_prep_body)

_gather = functools.partial(
    pl.kernel,
    out_type=jax.ShapeDtypeStruct((ROWS, HW), jnp.float32),
    mesh=_mesh,
    compiler_params=_params,
    scratch_types=[
        pltpu.VMEM((V,), jnp.float32),
        pltpu.VMEM((V,), jnp.float32),
        pltpu.VMEM((PCH,), jnp.int32),
        pltpu.VMEM((PCH,), jnp.int32),
        pltpu.VMEM((PCH,), jnp.float32),
        pltpu.VMEM((PCH,), jnp.float32),
        pltpu.VMEM((PCH,), jnp.float32),
        pltpu.VMEM((PCH,), jnp.float32),
        pltpu.SemaphoreType.DMA,
        pltpu.SemaphoreType.DMA,
        pltpu.SemaphoreType.DMA,
        pltpu.SemaphoreType.DMA,
    ],
)(_gather_body)


@jax.jit
def kernel(vertex_values, indices, v2i_idx):
    ind32 = indices.astype(jnp.int32)
    v2i = v2i_idx.astype(jnp.int32)
    cidx, tblt = _prep(vertex_values, ind32, v2i)
    out = _gather(tblt, cidx)
    return out.reshape(B, C, H, W)


# speedup vs baseline: 1.0024x; 1.0024x over previous
import functools

import jax
import jax.numpy as jnp
from jax import lax
from jax.experimental import pallas as pl
from jax.experimental.pallas import tpu as pltpu
from jax.experimental.pallas import tpu_sc as plsc

B, V, C = 8, 50000, 16
H = W = 512
HW = H * W

NC, NS = 2, 16
NW = NC * NS
LANES = 16

PIX_PER_W = HW // NW
PIX_SUB = 4096
TBLK = 2000
TPITCH = TBLK + 1
N_TTASK = B * (V // TBLK)
TTASK_PER_W = (N_TTASK + NW - 1) // NW

ROWS = B * C
ROW_PER_W = ROWS // NW
PCH = 4096
N_PCH = HW // PCH

_mesh = plsc.VectorSubcoreMesh(core_axis_name="c", subcore_axis_name="s")


def _wid():
    return lax.axis_index("s") * NC + lax.axis_index("c")


def _prep_body(vv_hbm, ind_hbm, v2i_hbm, cidx_hbm, tblt_hbm,
               ind_v, v2i_v, cidx_v, tin_v, tcol_v):
    w = _wid()

    pltpu.sync_copy(ind_hbm, ind_v)
    for sub in range(PIX_PER_W // PIX_SUB):
        poff = w * PIX_PER_W + sub * PIX_SUB
        row0 = poff // W
        pltpu.sync_copy(v2i_hbm.at[pl.ds(row0, PIX_SUB // W), :], v2i_v)

        @plsc.parallel_loop(0, PIX_SUB // LANES, unroll=8)
        def _(j):
            idx = v2i_v[j // (W // LANES), pl.ds((j % (W // LANES)) * LANES,
                                                 LANES)]
            cidx_v[pl.ds(j * LANES, LANES)] = plsc.load_gather(ind_v, [idx])
        pltpu.sync_copy(cidx_v, cidx_hbm.at[pl.ds(poff, PIX_SUB)])

    iota = lax.iota(jnp.int32, LANES)

    def ttask(t):
        b = t // (V // TBLK)
        roff = (t % (V // TBLK)) * TBLK
        pltpu.sync_copy(vv_hbm.at[b, pl.ds(roff, TBLK), :], tin_v)

        @plsc.parallel_loop(0, TBLK, unroll=8)
        def _(p):
            val = tin_v[p, :]
            plsc.store_scatter(
                tcol_v, [iota, jnp.full((LANES,), p, jnp.int32)], val)
        pltpu.sync_copy(tcol_v.at[:, pl.ds(0, TBLK)],
                        tblt_hbm.at[pl.ds(b * C, C), pl.ds(roff, TBLK)])

    def touter(k, _):
        t = w + k * NW

        @pl.when(t < N_TTASK)
        def _():
            ttask(t)

        return 0

    lax.fori_loop(0, TTASK_PER_W, touter, 0)


def _gather_body(tblt_hbm, cidx_hbm, out_hbm,
                 ta_v, tb_v, c0_v, c1_v, o00_v, o10_v, o01_v, o11_v,
                 si0, si1, so0, so1):
    w = _wid()
    cbufs = ((c0_v, si0), (c1_v, si1))
    obufs = ((o00_v, o10_v, so0), (o01_v, o11_v, so1))

    for half in range(ROW_PER_W // 2):
        r0 = w * ROW_PER_W + half * 2
        pltpu.sync_copy(tblt_hbm.at[r0], ta_v)
        pltpu.sync_copy(tblt_hbm.at[r0 + 1], tb_v)
        pltpu.async_copy(cidx_hbm.at[pl.ds(0, PCH)], c0_v, si0)

        def sub(kk, i):
            cin, si = cbufs[i]
            oa, ob, so = obufs[i]
            cnx, snx = cbufs[1 - i]
            ch = kk * 2 + i
            pltpu.make_async_copy(cidx_hbm.at[pl.ds(0, PCH)], cin, si).wait()

            @pl.when(ch + 1 < N_PCH)
            def _():
                pltpu.async_copy(
                    cidx_hbm.at[pl.ds((ch + 1) * PCH, PCH)], cnx, snx)

            @pl.when(kk > 0)
            def _():
                pltpu.make_async_copy(
                    oa, out_hbm.at[r0, pl.ds(0, PCH)], so).wait()
                pltpu.make_async_copy(
                    ob, out_hbm.at[r0 + 1, pl.ds(0, PCH)], so).wait()

            @plsc.parallel_loop(0, PCH // LANES, unroll=8)
            def _(j):
                s = pl.ds(j * LANES, LANES)
                idx = cin[s]
                oa[s] = plsc.load_gather(ta_v, [idx])
                ob[s] = plsc.load_gather(tb_v, [idx])

            poff = ch * PCH
            pltpu.async_copy(oa, out_hbm.at[r0, pl.ds(poff, PCH)], so)
            pltpu.async_copy(ob, out_hbm.at[r0 + 1, pl.ds(poff, PCH)], so)

        def kk_body(kk, _):
            sub(kk, 0)
            sub(kk, 1)
            return 0

        lax.fori_loop(0, N_PCH // 2, kk_body, 0)
        for oa, ob, so in obufs:
            pltpu.make_async_copy(oa, out_hbm.at[r0, pl.ds(0, PCH)], so).wait()
            pltpu.make_async_copy(
                ob, out_hbm.at[r0 + 1, pl.ds(0, PCH)], so).wait()


_params = pltpu.CompilerParams(use_tc_tiling_on_sc=False,
                               needs_layout_passes=False)

_prep = functools.partial(
    pl.kernel,
    out_type=(
        jax.ShapeDtypeStruct((HW,), jnp.int32),
        jax.ShapeDtypeStruct((B * C, V), jnp.float32),
    ),
    mesh=_mesh,
    compiler_params=_params,
    scratch_types=[
        pltpu.VMEM((V,), jnp.int32),
        pltpu.VMEM((PIX_SUB // W, W), jnp.int32),
        pltpu.VMEM((PIX_SUB,), jnp.int32),
        pltpu.VMEM((TBLK, C), jnp.float32),
        pltpu.VMEM((C, TPITCH), jnp.float32),
    ],
)(_prep_body)

_gather = functools.partial(
    pl.kernel,
    out_type=jax.ShapeDtypeStruct((ROWS, HW), jnp.float32),
    mesh=_mesh,
    compiler_params=_params,
    scratch_types=[
        pltpu.VMEM((V,), jnp.float32),
        pltpu.VMEM((V,), jnp.float32),
        pltpu.VMEM((PCH,), jnp.int32),
        pltpu.VMEM((PCH,), jnp.int32),
        pltpu.VMEM((PCH,), jnp.float32),
        pltpu.VMEM((PCH,), jnp.float32),
        pltpu.VMEM((PCH,), jnp.float32),
        pltpu.VMEM((PCH,), jnp.float32),
        pltpu.SemaphoreType.DMA,
        pltpu.SemaphoreType.DMA,
        pltpu.SemaphoreType.DMA,
        pltpu.SemaphoreType.DMA,
    ],
)(_gather_body)


@jax.jit
def kernel(vertex_values, indices, v2i_idx):
    ind32 = indices if indices.dtype == jnp.int32 else indices.astype(jnp.int32)
    v2i = v2i_idx if v2i_idx.dtype == jnp.int32 else v2i_idx.astype(jnp.int32)
    cidx, tblt = _prep(vertex_values, ind32, v2i)
    out = _gather(tblt, cidx)
    return out.reshape(B, C, H, W)


# speedup vs baseline: 1.0036x; 1.0012x over previous
import functools

import jax
import jax.numpy as jnp
from jax import lax
from jax.experimental import layout as jlayout
from jax.experimental import pallas as pl
from jax.experimental.pallas import tpu as pltpu
from jax.experimental.pallas import tpu_sc as plsc

B, V, C = 8, 50000, 16
H = W = 512
HW = H * W

NC, NS = 2, 16
NW = NC * NS
LANES = 16

PIX_PER_W = HW // NW
PIX_SUB = 4096
TBLK = 2000
TPITCH = TBLK + 1
N_TTASK = B * (V // TBLK)
TTASK_PER_W = (N_TTASK + NW - 1) // NW

ROWS = B * C
ROW_PER_W = ROWS // NW
PCH = 4096
N_PCH = HW // PCH

_mesh = plsc.VectorSubcoreMesh(core_axis_name="c", subcore_axis_name="s")


def _wid():
    return lax.axis_index("s") * NC + lax.axis_index("c")


def _prep_body(vv_hbm, ind_hbm, v2i_hbm, cidx_hbm, tblt_hbm,
               ind_v, v2i_v, cidx_v, tin_v, tcol_v):
    w = _wid()

    pltpu.sync_copy(ind_hbm, ind_v)
    for sub in range(PIX_PER_W // PIX_SUB):
        poff = w * PIX_PER_W + sub * PIX_SUB
        row0 = poff // W
        pltpu.sync_copy(v2i_hbm.at[pl.ds(row0, PIX_SUB // W), :], v2i_v)

        @plsc.parallel_loop(0, PIX_SUB // LANES, unroll=8)
        def _(j):
            idx = v2i_v[j // (W // LANES), pl.ds((j % (W // LANES)) * LANES,
                                                 LANES)]
            cidx_v[pl.ds(j * LANES, LANES)] = plsc.load_gather(ind_v, [idx])
        pltpu.sync_copy(cidx_v, cidx_hbm.at[pl.ds(poff, PIX_SUB)])

    iota = lax.iota(jnp.int32, LANES)

    def ttask(t):
        b = t // (V // TBLK)
        roff = (t % (V // TBLK)) * TBLK
        pltpu.sync_copy(vv_hbm.at[b, pl.ds(roff, TBLK), :], tin_v)

        @plsc.parallel_loop(0, TBLK, unroll=8)
        def _(p):
            val = tin_v[p, :]
            plsc.store_scatter(
                tcol_v, [iota, jnp.full((LANES,), p, jnp.int32)], val)
        pltpu.sync_copy(tcol_v.at[:, pl.ds(0, TBLK)],
                        tblt_hbm.at[pl.ds(b * C, C), pl.ds(roff, TBLK)])

    def touter(k, _):
        t = w + k * NW

        @pl.when(t < N_TTASK)
        def _():
            ttask(t)

        return 0

    lax.fori_loop(0, TTASK_PER_W, touter, 0)


def _gather_body(tblt_hbm, cidx_hbm, out_hbm,
                 ta_v, tb_v, c0_v, c1_v, o00_v, o10_v, o01_v, o11_v,
                 si0, si1, so0, so1):
    w = _wid()
    cbufs = ((c0_v, si0), (c1_v, si1))
    obufs = ((o00_v, o10_v, so0), (o01_v, o11_v, so1))

    for half in range(ROW_PER_W // 2):
        r0 = w * ROW_PER_W + half * 2
        pltpu.sync_copy(tblt_hbm.at[r0], ta_v)
        pltpu.sync_copy(tblt_hbm.at[r0 + 1], tb_v)
        pltpu.async_copy(cidx_hbm.at[pl.ds(0, PCH)], c0_v, si0)

        def sub(kk, i):
            cin, si = cbufs[i]
            oa, ob, so = obufs[i]
            cnx, snx = cbufs[1 - i]
            ch = kk * 2 + i
            pltpu.make_async_copy(cidx_hbm.at[pl.ds(0, PCH)], cin, si).wait()

            @pl.when(ch + 1 < N_PCH)
            def _():
                pltpu.async_copy(
                    cidx_hbm.at[pl.ds((ch + 1) * PCH, PCH)], cnx, snx)

            @pl.when(kk > 0)
            def _():
                pltpu.make_async_copy(
                    oa, out_hbm.at[r0, pl.ds(0, PCH)], so).wait()
                pltpu.make_async_copy(
                    ob, out_hbm.at[r0 + 1, pl.ds(0, PCH)], so).wait()

            @plsc.parallel_loop(0, PCH // LANES, unroll=8)
            def _(j):
                s = pl.ds(j * LANES, LANES)
                idx = cin[s]
                oa[s] = plsc.load_gather(ta_v, [idx])
                ob[s] = plsc.load_gather(tb_v, [idx])

            poff = ch * PCH
            pltpu.async_copy(oa, out_hbm.at[r0, pl.ds(poff, PCH)], so)
            pltpu.async_copy(ob, out_hbm.at[r0 + 1, pl.ds(poff, PCH)], so)

        def kk_body(kk, _):
            sub(kk, 0)
            sub(kk, 1)
            return 0

        lax.fori_loop(0, N_PCH // 2, kk_body, 0)
        for oa, ob, so in obufs:
            pltpu.make_async_copy(oa, out_hbm.at[r0, pl.ds(0, PCH)], so).wait()
            pltpu.make_async_copy(
                ob, out_hbm.at[r0 + 1, pl.ds(0, PCH)], so).wait()


_params = pltpu.CompilerParams(use_tc_tiling_on_sc=False,
                               needs_layout_passes=False)

_prep = functools.partial(
    pl.kernel,
    out_type=(
        jax.ShapeDtypeStruct((HW,), jnp.int32),
        jax.ShapeDtypeStruct((B * C, V), jnp.float32),
    ),
    mesh=_mesh,
    compiler_params=_params,
    scratch_types=[
        pltpu.VMEM((V,), jnp.int32),
        pltpu.VMEM((PIX_SUB // W, W), jnp.int32),
        pltpu.VMEM((PIX_SUB,), jnp.int32),
        pltpu.VMEM((TBLK, C), jnp.float32),
        pltpu.VMEM((C, TPITCH), jnp.float32),
    ],
)(_prep_body)

_gather = functools.partial(
    pl.kernel,
    out_type=jax.ShapeDtypeStruct((ROWS, HW), jnp.float32),
    mesh=_mesh,
    compiler_params=_params,
    scratch_types=[
        pltpu.VMEM((V,), jnp.float32),
        pltpu.VMEM((V,), jnp.float32),
        pltpu.VMEM((PCH,), jnp.int32),
        pltpu.VMEM((PCH,), jnp.int32),
        pltpu.VMEM((PCH,), jnp.float32),
        pltpu.VMEM((PCH,), jnp.float32),
        pltpu.VMEM((PCH,), jnp.float32),
        pltpu.VMEM((PCH,), jnp.float32),
        pltpu.SemaphoreType.DMA,
        pltpu.SemaphoreType.DMA,
        pltpu.SemaphoreType.DMA,
        pltpu.SemaphoreType.DMA,
    ],
)(_gather_body)


def _impl(vertex_values, indices, v2i_idx):
    ind32 = indices if indices.dtype == jnp.int32 else indices.astype(jnp.int32)
    v2i = v2i_idx if v2i_idx.dtype == jnp.int32 else v2i_idx.astype(jnp.int32)
    cidx, tblt = _prep(vertex_values, ind32, v2i)
    out = _gather(tblt, cidx)
    return out.reshape(B, C, H, W)


@functools.cache
def _jitted():
    fmt = jlayout.Format(
        jlayout.Layout(major_to_minor=(0, 1, 2, 3), tiling=()),
        jax.sharding.SingleDeviceSharding(jax.devices()[0]))
    return jax.jit(_impl, out_shardings=fmt)


def kernel(vertex_values, indices, v2i_idx):
    return _jitted()(vertex_values, indices, v2i_idx)
